# baseline (speedup 1.0000x reference)
import functools

import jax
import jax.numpy as jnp
from jax import lax
from jax.experimental import pallas as pl
from jax.experimental.pallas import tpu as pltpu
from jax.experimental.pallas import tpu_sc as plsc

_TC_CHUNKS = 4
_SC_CH = 32


def _tc_make_body(b, n, d, n_chunks):
    rows = n // n_chunks

    def body(pe_hbm, out_hbm, vmem, in_sems, out_sems):
        for c in range(n_chunks):
            sl = pl.ds(c * rows, rows)
            pltpu.make_async_copy(pe_hbm.at[sl], vmem.at[sl], in_sems.at[c]).start()
        for c in range(n_chunks):
            sl = pl.ds(c * rows, rows)
            pltpu.make_async_copy(pe_hbm.at[sl], vmem.at[sl], in_sems.at[c]).wait()
            for i in range(b):
                pltpu.make_async_copy(
                    vmem.at[sl], out_hbm.at[i, sl], out_sems.at[c, i]
                ).start()
        for c in range(n_chunks):
            sl = pl.ds(c * rows, rows)
            for i in range(b):
                pltpu.make_async_copy(
                    vmem.at[sl], out_hbm.at[i, sl], out_sems.at[c, i]
                ).wait()

    return body


def _tc_call(pe_weight, b, n, d):
    return pl.pallas_call(
        _tc_make_body(b, n, d, _TC_CHUNKS),
        in_specs=[pl.BlockSpec(memory_space=pl.ANY)],
        out_specs=pl.BlockSpec(memory_space=pl.ANY),
        out_shape=jax.ShapeDtypeStruct((b, n, d), pe_weight.dtype),
        scratch_shapes=[
            pltpu.VMEM((n, d), pe_weight.dtype),
            pltpu.SemaphoreType.DMA((_TC_CHUNKS,)),
            pltpu.SemaphoreType.DMA((_TC_CHUNKS, b)),
        ],
    )(pe_weight)


def _sc_call(pe_weight, n, d):
    dtype = pe_weight.dtype
    mesh = plsc.VectorSubcoreMesh(core_axis_name="c", subcore_axis_name="s")
    nw = mesh.num_cores * mesh.num_subcores
    rows_w = n // nw
    nch = rows_w // _SC_CH

    @functools.partial(
        pl.kernel,
        out_type=jax.ShapeDtypeStruct((1, n, d), dtype),
        mesh=mesh,
        scratch_types=[
            pltpu.VMEM((2, _SC_CH, d), dtype),
            pltpu.SemaphoreType.DMA((nch,)),
            pltpu.SemaphoreType.DMA((nch,)),
        ],
    )
    def sc_copy(pe_hbm, out_hbm, bufs, in_sems, out_sems):
        wid = lax.axis_index("s") * mesh.num_cores + lax.axis_index("c")
        base = wid * rows_w

        def sl(c):
            return pl.ds(base + c * _SC_CH, _SC_CH)

        pltpu.make_async_copy(pe_hbm.at[sl(0)], bufs.at[0], in_sems.at[0]).start()
        if nch > 1:
            pltpu.make_async_copy(pe_hbm.at[sl(1)], bufs.at[1], in_sems.at[1]).start()
        for c in range(nch):
            buf = bufs.at[c % 2]
            pltpu.make_async_copy(pe_hbm.at[sl(c)], buf, in_sems.at[c]).wait()
            pltpu.make_async_copy(buf, out_hbm.at[0, sl(c)], out_sems.at[c]).start()
            if c + 2 < nch:
                pltpu.make_async_copy(
                    bufs.at[c % 2], out_hbm.at[0, sl(c)], out_sems.at[c]
                ).wait()
                pltpu.make_async_copy(
                    pe_hbm.at[sl(c + 2)], bufs.at[c % 2], in_sems.at[c + 2]
                ).start()
        for c in range(max(0, nch - 2), nch):
            pltpu.make_async_copy(
                bufs.at[c % 2], out_hbm.at[0, sl(c)], out_sems.at[c]
            ).wait()

    return sc_copy(pe_weight)


def kernel(x, pe_weight):
    b = x.shape[0]
    n, d = pe_weight.shape
    tc_part = _tc_call(pe_weight, b - 1, n, d)
    sc_part = _sc_call(pe_weight, n, d)
    return jnp.concatenate([tc_part, sc_part], axis=0)


# speedup vs baseline: 3.3329x; 3.3329x over previous
import jax
import jax.numpy as jnp
from jax.experimental import pallas as pl
from jax.experimental.pallas import tpu as pltpu

_N_CHUNKS = 4


def _make_body(b, n, d, n_chunks):
    rows = n // n_chunks

    def body(pe_hbm, out_hbm, vmem, in_sems, out_sems):
        for c in range(n_chunks):
            sl = pl.ds(c * rows, rows)
            pltpu.make_async_copy(pe_hbm.at[sl], vmem.at[sl], in_sems.at[c]).start()
        for c in range(n_chunks):
            sl = pl.ds(c * rows, rows)
            pltpu.make_async_copy(pe_hbm.at[sl], vmem.at[sl], in_sems.at[c]).wait()
        for i in range(b):
            pltpu.make_async_copy(vmem, out_hbm.at[i], out_sems.at[i]).start()
        for i in range(b):
            pltpu.make_async_copy(vmem, out_hbm.at[i], out_sems.at[i]).wait()

    return body


def kernel(x, pe_weight):
    b = x.shape[0]
    n, d = pe_weight.shape
    n_chunks = _N_CHUNKS if n % _N_CHUNKS == 0 else 1
    return pl.pallas_call(
        _make_body(b, n, d, n_chunks),
        in_specs=[pl.BlockSpec(memory_space=pl.ANY)],
        out_specs=pl.BlockSpec(memory_space=pl.ANY),
        out_shape=jax.ShapeDtypeStruct((b, n, d), pe_weight.dtype),
        scratch_shapes=[
            pltpu.VMEM((n, d), pe_weight.dtype),
            pltpu.SemaphoreType.DMA((n_chunks,)),
            pltpu.SemaphoreType.DMA((b,)),
        ],
    )(pe_weight)


# speedup vs baseline: 3.6405x; 1.0923x over previous
import jax
import jax.numpy as jnp
from jax.experimental import pallas as pl
from jax.experimental.pallas import tpu as pltpu

_N_CHUNKS = 4


def _make_body(b, n, d, n_chunks):
    rows = n // n_chunks

    def body(pe_hbm, out_hbm, vmem, in_sems, out_sems):
        for c in range(n_chunks):
            sl = pl.ds(c * rows, rows)
            pltpu.make_async_copy(pe_hbm.at[sl], vmem.at[sl], in_sems.at[c]).start()
        half = rows // 2
        for c in range(n_chunks):
            sl = pl.ds(c * rows, rows)
            pltpu.make_async_copy(pe_hbm.at[sl], vmem.at[sl], in_sems.at[c]).wait()
            for i in range(b):
                for h in range(2):
                    hs = pl.ds(c * rows + h * half, half)
                    pltpu.make_async_copy(
                        vmem.at[hs], out_hbm.at[i, hs], out_sems.at[c, i]
                    ).start()
        for c in range(n_chunks):
            for i in range(b):
                for h in range(2):
                    hs = pl.ds(c * rows + h * half, half)
                    pltpu.make_async_copy(
                        vmem.at[hs], out_hbm.at[i, hs], out_sems.at[c, i]
                    ).wait()

    return body


def kernel(x, pe_weight):
    b = x.shape[0]
    n, d = pe_weight.shape
    n_chunks = _N_CHUNKS if n % _N_CHUNKS == 0 else 1
    return pl.pallas_call(
        _make_body(b, n, d, n_chunks),
        in_specs=[pl.BlockSpec(memory_space=pl.ANY)],
        out_specs=pl.BlockSpec(memory_space=pl.ANY),
        out_shape=jax.ShapeDtypeStruct((b, n, d), pe_weight.dtype),
        scratch_shapes=[
            pltpu.VMEM((n, d), pe_weight.dtype),
            pltpu.SemaphoreType.DMA((n_chunks,)),
            pltpu.SemaphoreType.DMA((_N_CHUNKS, b)),
        ],
    )(pe_weight)
